# baseline (speedup 1.0000x reference)
import functools

import jax
import jax.numpy as jnp
from jax import lax
from jax.experimental import pallas as pl
from jax.experimental.pallas import tpu as pltpu
from jax.experimental.pallas import tpu_sc as plsc

USER_NUM = 52643
ITEM_NUM = 91599
D = 64
DP = 128
B = 16384

NW = 32

BLK = 512
UPAD = 52736
IPAD = 91648
UBLOCKS = UPAD // BLK
IBLOCKS = IPAD // BLK

BPW = B // NW
HALF = BPW // 2
CHUNK = 128
NCHUNK = HALF // CHUNK
GROUPS = HALF // 16

_mesh = plsc.VectorSubcoreMesh(core_axis_name="c", subcore_axis_name="s")
_params = pltpu.CompilerParams(needs_layout_passes=False,
                               use_tc_tiling_on_sc=True,
                               disable_bounds_checks=True)


@functools.partial(
    pl.kernel,
    out_type=(
        jax.ShapeDtypeStruct((UPAD, DP), jnp.float32),
        jax.ShapeDtypeStruct((IPAD, DP), jnp.float32),
    ),
    mesh=_mesh,
    scratch_types=dict(
        in_buf=pltpu.VMEM((D, BLK), jnp.float32),
        out_stage=pltpu.VMEM((BLK, DP), jnp.float32),
        sem_in=pltpu.SemaphoreType.DMA,
        sem_out=pltpu.SemaphoreType.DMA,
    ),
    compiler_params=_params,
)
def _repack_kernel(ut_t, it_t, u_out, i_out, *,
                   in_buf, out_stage, sem_in, sem_out):
    wid = lax.axis_index("s") * 2 + lax.axis_index("c")
    lane = lax.iota(jnp.int32, 16)

    def do_block(tab_t, out, b):
        c0 = b * BLK
        pltpu.make_async_copy(tab_t.at[:, pl.ds(c0, BLK)], in_buf,
                              sem_in).start()
        pltpu.make_async_copy(tab_t.at[:, pl.ds(c0, BLK)], in_buf,
                              sem_in).wait()

        def micro(m, carry):
            j0 = m * 16
            rows = j0 + lane
            rot0 = (c0 + j0 + lane) & (D - 1)
            for d in range(D):
                v = in_buf[d, pl.ds(j0, 16)]
                col = (rot0 + d) & (D - 1)
                plsc.store_scatter(out_stage, [rows, col], v)
            return carry

        lax.fori_loop(0, BLK // 16, micro, 0)
        pltpu.make_async_copy(out_stage, out.at[pl.ds(c0, BLK)],
                              sem_out).start()
        pltpu.make_async_copy(out_stage, out.at[pl.ds(c0, BLK)],
                              sem_out).wait()

    for k in range(3):
        do_block(ut_t, u_out, wid + NW * k)

    @pl.when(wid + NW * 3 < UBLOCKS)
    def _():
        do_block(ut_t, u_out, wid + NW * 3)

    for k in range(5):
        do_block(it_t, i_out, wid + NW * k)

    @pl.when(wid + NW * 5 < IBLOCKS)
    def _():
        do_block(it_t, i_out, wid + NW * 5)


@functools.partial(
    pl.kernel,
    out_type=(
        jax.ShapeDtypeStruct((B,), jnp.float32),
        jax.ShapeDtypeStruct((B,), jnp.float32),
    ),
    mesh=_mesh,
    scratch_types=dict(
        idx_u=pltpu.VMEM((NCHUNK, CHUNK), jnp.int32),
        idx_p=pltpu.VMEM((NCHUNK, CHUNK), jnp.int32),
        idx_n=pltpu.VMEM((NCHUNK, CHUNK), jnp.int32),
        iflat_u=pltpu.VMEM((HALF,), jnp.int32),
        iflat_p=pltpu.VMEM((HALF,), jnp.int32),
        iflat_n=pltpu.VMEM((HALF,), jnp.int32),
        u_rows=pltpu.VMEM((HALF, DP), jnp.float32),
        p_rows=pltpu.VMEM((HALF, DP), jnp.float32),
        n_rows=pltpu.VMEM((HALF, DP), jnp.float32),
        p_loc=pltpu.VMEM((BPW,), jnp.float32),
        n_loc=pltpu.VMEM((BPW,), jnp.float32),
        sem_idx=pltpu.SemaphoreType.DMA,
        sem_rows=pltpu.SemaphoreType.DMA,
    ),
    compiler_params=_params,
)
def _score_kernel(users, positives, negatives, user_table, item_table,
                  p_out, n_out, *, idx_u, idx_p, idx_n,
                  iflat_u, iflat_p, iflat_n,
                  u_rows, p_rows, n_rows, p_loc, n_loc, sem_idx, sem_rows):
    wid = lax.axis_index("s") * 2 + lax.axis_index("c")
    base = wid * BPW
    lane = lax.iota(jnp.int32, 16)

    for h in range(2):
        hbase = base + h * HALF

        idx_copies = []
        for j in range(NCHUNK):
            for src, dst in ((users, idx_u), (positives, idx_p),
                             (negatives, idx_n)):
                c = pltpu.make_async_copy(
                    src.at[pl.ds(hbase + j * CHUNK, CHUNK)], dst.at[j],
                    sem_idx)
                c.start()
                idx_copies.append(c)
        for src, dst in ((users, iflat_u), (positives, iflat_p),
                         (negatives, iflat_n)):
            c = pltpu.make_async_copy(src.at[pl.ds(hbase, HALF)], dst,
                                      sem_idx)
            c.start()
            idx_copies.append(c)
        for c in idx_copies:
            c.wait()

        row_copies = []
        for j in range(NCHUNK):
            for tab, idx, dst in ((user_table, idx_u, u_rows),
                                  (item_table, idx_p, p_rows),
                                  (item_table, idx_n, n_rows)):
                c = pltpu.make_async_copy(
                    tab.at[idx.at[j]], dst.at[pl.ds(j * CHUNK, CHUNK)],
                    sem_rows)
                c.start()
                row_copies.append(c)
        for c in row_copies:
            c.wait()

        def group_body(g, carry):
            rows = g * 16 + lane
            cbu = iflat_u[pl.ds(g * 16, 16)] & (D - 1)
            cbp = iflat_p[pl.ds(g * 16, 16)] & (D - 1)
            cbn = iflat_n[pl.ds(g * 16, 16)] & (D - 1)
            accp = jnp.zeros((16,), jnp.float32)
            accn = jnp.zeros((16,), jnp.float32)
            for d in range(D):
                u = plsc.load_gather(u_rows, [rows, (cbu + d) & (D - 1)])
                pv = plsc.load_gather(p_rows, [rows, (cbp + d) & (D - 1)])
                nv = plsc.load_gather(n_rows, [rows, (cbn + d) & (D - 1)])
                accp = accp + u * pv
                accn = accn + u * nv
            p_loc[pl.ds(h * HALF + g * 16, 16)] = accp
            n_loc[pl.ds(h * HALF + g * 16, 16)] = accn
            return carry

        lax.fori_loop(0, GROUPS, group_body, 0)

    pltpu.sync_copy(p_loc, p_out.at[pl.ds(base, BPW)])
    pltpu.sync_copy(n_loc, n_out.at[pl.ds(base, BPW)])


def kernel(users, positives, negatives, user_table, item_table):
    utp, itp = _repack_kernel(user_table.T, item_table.T)
    return _score_kernel(users.astype(jnp.int32), positives.astype(jnp.int32),
                         negatives.astype(jnp.int32), utp, itp)


# speedup vs baseline: 1.4005x; 1.4005x over previous
import functools

import jax
import jax.numpy as jnp
from jax import lax
from jax.experimental import pallas as pl
from jax.experimental.pallas import tpu as pltpu
from jax.experimental.pallas import tpu_sc as plsc

USER_NUM = 52643
ITEM_NUM = 91599
D = 64
DP = 128
B = 16384

NW = 32

BLK = 256
UPAD = 52736
IPAD = 91648
UBLOCKS = UPAD // BLK
IBLOCKS = IPAD // BLK
UPAIRS = -(-UBLOCKS // (2 * NW))
IPAIRS = -(-IBLOCKS // (2 * NW))

BPW = B // NW
HALF = BPW // 2
CHUNK = 128
NCHUNK = HALF // CHUNK
GROUPS = HALF // 16

_mesh = plsc.VectorSubcoreMesh(core_axis_name="c", subcore_axis_name="s")
_params = pltpu.CompilerParams(needs_layout_passes=False,
                               use_tc_tiling_on_sc=True,
                               disable_bounds_checks=True)


@functools.partial(
    pl.kernel,
    out_type=(
        jax.ShapeDtypeStruct((UPAD, DP), jnp.float32),
        jax.ShapeDtypeStruct((IPAD, DP), jnp.float32),
    ),
    mesh=_mesh,
    scratch_types=dict(
        in_buf0=pltpu.VMEM((D, BLK), jnp.float32),
        in_buf1=pltpu.VMEM((D, BLK), jnp.float32),
        out_stage0=pltpu.VMEM((BLK, DP), jnp.float32),
        out_stage1=pltpu.VMEM((BLK, DP), jnp.float32),
        sem_in0=pltpu.SemaphoreType.DMA,
        sem_in1=pltpu.SemaphoreType.DMA,
        sem_out0=pltpu.SemaphoreType.DMA,
        sem_out1=pltpu.SemaphoreType.DMA,
    ),
    compiler_params=_params,
)
def _repack_kernel(ut_t, it_t, u_out, i_out, *,
                   in_buf0, in_buf1, out_stage0, out_stage1,
                   sem_in0, sem_in1, sem_out0, sem_out1):
    wid = lax.axis_index("s") * 2 + lax.axis_index("c")
    lane = lax.iota(jnp.int32, 16)
    in_bufs = (in_buf0, in_buf1)
    out_stages = (out_stage0, out_stage1)
    sem_ins = (sem_in0, sem_in1)
    sem_outs = (sem_out0, sem_out1)

    def in_copy(tab_t, k, slot):
        c0 = (wid + NW * k) * BLK
        return pltpu.make_async_copy(tab_t.at[:, pl.ds(c0, BLK)],
                                     in_bufs[slot], sem_ins[slot])

    def out_copy(out, k, slot):
        c0 = (wid + NW * k) * BLK
        return pltpu.make_async_copy(
            out_stages[slot], out.at[pl.ds(c0, BLK)],
            sem_outs[slot])

    def compute(k, slot):
        c0 = (wid + NW * k) * BLK
        src = in_bufs[slot]
        dst = out_stages[slot]

        def micro(m, carry):
            j0 = m * 16
            rows = j0 + lane
            rot0 = (c0 + j0 + lane) & (D - 1)
            for d in range(D):
                v = src[d, pl.ds(j0, 16)]
                col = (rot0 + d) & (D - 1)
                plsc.store_scatter(dst, [rows, col], v)
            return carry

        lax.fori_loop(0, BLK // 16, micro, 0)

    def phase(tab_t, out, npairs, nblocks):
        nrounds = 2 * npairs

        def valid(k):
            return wid + NW * k < nblocks

        for s in range(2):
            @pl.when(valid(s))
            def _(s=s):
                in_copy(tab_t, s, s).start()

        def pair(kp, carry):
            for s in range(2):
                k = 2 * kp + s

                @pl.when(valid(k))
                def _(k=k, s=s):
                    in_copy(tab_t, k, s).wait()

                    @pl.when(kp >= 1)
                    def _():
                        out_copy(out, k - 2, s).wait()

                    compute(k, s)
                    out_copy(out, k, s).start()

                @pl.when(valid(k + 2))
                def _(k=k, s=s):
                    in_copy(tab_t, k + 2, s).start()
            return carry

        lax.fori_loop(0, npairs, pair, 0)

        for j in range(max(nrounds - 3, 0), nrounds):
            later = valid(j + 2) if j + 2 < nrounds else False

            @pl.when(valid(j) & jnp.logical_not(later))
            def _(j=j):
                out_copy(out, j, j % 2).wait()

    phase(ut_t, u_out, UPAIRS, UBLOCKS)
    phase(it_t, i_out, IPAIRS, IBLOCKS)


@functools.partial(
    pl.kernel,
    out_type=(
        jax.ShapeDtypeStruct((B,), jnp.float32),
        jax.ShapeDtypeStruct((B,), jnp.float32),
    ),
    mesh=_mesh,
    scratch_types=dict(
        idx_u=pltpu.VMEM((NCHUNK, CHUNK), jnp.int32),
        idx_p=pltpu.VMEM((NCHUNK, CHUNK), jnp.int32),
        idx_n=pltpu.VMEM((NCHUNK, CHUNK), jnp.int32),
        iflat_u=pltpu.VMEM((HALF,), jnp.int32),
        iflat_p=pltpu.VMEM((HALF,), jnp.int32),
        iflat_n=pltpu.VMEM((HALF,), jnp.int32),
        u_rows=pltpu.VMEM((HALF, DP), jnp.float32),
        p_rows=pltpu.VMEM((HALF, DP), jnp.float32),
        n_rows=pltpu.VMEM((HALF, DP), jnp.float32),
        p_loc=pltpu.VMEM((BPW,), jnp.float32),
        n_loc=pltpu.VMEM((BPW,), jnp.float32),
        sem_idx=pltpu.SemaphoreType.DMA,
        sem_rows=pltpu.SemaphoreType.DMA,
    ),
    compiler_params=_params,
)
def _score_kernel(users, positives, negatives, user_table, item_table,
                  p_out, n_out, *, idx_u, idx_p, idx_n,
                  iflat_u, iflat_p, iflat_n,
                  u_rows, p_rows, n_rows, p_loc, n_loc, sem_idx, sem_rows):
    wid = lax.axis_index("s") * 2 + lax.axis_index("c")
    base = wid * BPW
    lane = lax.iota(jnp.int32, 16)

    for h in range(2):
        hbase = base + h * HALF

        idx_copies = []
        for j in range(NCHUNK):
            for src, dst in ((users, idx_u), (positives, idx_p),
                             (negatives, idx_n)):
                c = pltpu.make_async_copy(
                    src.at[pl.ds(hbase + j * CHUNK, CHUNK)], dst.at[j],
                    sem_idx)
                c.start()
                idx_copies.append(c)
        for src, dst in ((users, iflat_u), (positives, iflat_p),
                         (negatives, iflat_n)):
            c = pltpu.make_async_copy(src.at[pl.ds(hbase, HALF)], dst,
                                      sem_idx)
            c.start()
            idx_copies.append(c)
        for c in idx_copies:
            c.wait()

        row_copies = []
        for j in range(NCHUNK):
            for tab, idx, dst in ((user_table, idx_u, u_rows),
                                  (item_table, idx_p, p_rows),
                                  (item_table, idx_n, n_rows)):
                c = pltpu.make_async_copy(
                    tab.at[idx.at[j]], dst.at[pl.ds(j * CHUNK, CHUNK)],
                    sem_rows)
                c.start()
                row_copies.append(c)
        for c in row_copies:
            c.wait()

        def group_body(g, carry):
            rows = g * 16 + lane
            cbu = iflat_u[pl.ds(g * 16, 16)] & (D - 1)
            cbp = iflat_p[pl.ds(g * 16, 16)] & (D - 1)
            cbn = iflat_n[pl.ds(g * 16, 16)] & (D - 1)
            accp = jnp.zeros((16,), jnp.float32)
            accn = jnp.zeros((16,), jnp.float32)
            for d in range(D):
                u = plsc.load_gather(u_rows, [rows, (cbu + d) & (D - 1)])
                pv = plsc.load_gather(p_rows, [rows, (cbp + d) & (D - 1)])
                nv = plsc.load_gather(n_rows, [rows, (cbn + d) & (D - 1)])
                accp = accp + u * pv
                accn = accn + u * nv
            p_loc[pl.ds(h * HALF + g * 16, 16)] = accp
            n_loc[pl.ds(h * HALF + g * 16, 16)] = accn
            return carry

        lax.fori_loop(0, GROUPS, group_body, 0)

    pltpu.sync_copy(p_loc, p_out.at[pl.ds(base, BPW)])
    pltpu.sync_copy(n_loc, n_out.at[pl.ds(base, BPW)])


def kernel(users, positives, negatives, user_table, item_table):
    utp, itp = _repack_kernel(user_table.T, item_table.T)
    return _score_kernel(users.astype(jnp.int32), positives.astype(jnp.int32),
                         negatives.astype(jnp.int32), utp, itp)
